# baseline (speedup 1.0000x reference)
import functools

import numpy as np

import jax
import jax.numpy as jnp
from jax import lax
from jax.experimental import pallas as pl
from jax.experimental.pallas import tpu as pltpu
from jax.experimental.pallas import tpu_sc as plsc

E = 160000
N = 10000
D = 256
DH = 128
R = 16

NC = 2
NS = 16
EP = E // NS
EPH = EP // 2
GS = 80
GSH = GS // 2
G = EP // GS
WB = 624
WBT = N - NS * WB
ZR = 48

EBLK = 8000
HB = EBLK // 2
NBLK = 1000


def _edge_fma_body(m_ref, rbf_ref, w_ref, out_ref):
    mlp = jnp.dot(rbf_ref[...], w_ref[...], preferred_element_type=jnp.float32)
    y = (m_ref[...] * mlp).astype(jnp.bfloat16)
    u = lax.bitcast_convert_type(y, jnp.uint16)
    lo = u[:HB].astype(jnp.uint32)
    hi = u[HB:].astype(jnp.uint32)
    packed = lax.bitcast_convert_type(lo | (hi << 16), jnp.int32)
    out_ref[0] = packed[:, :DH]
    out_ref[1] = packed[:, DH:]


def _seg_sum_body(x_hbm, idx_hbm, out_hbm, idx_v, xint, xf32, gsem, ssem,
                  accum):
    c = lax.axis_index("c")
    s = lax.axis_index("s")
    himask = jnp.int32(-65536)

    for i in range(ZR):
        for k in range(DH // 16):
            xf32[0, i, pl.ds(k * 16, 16)] = jnp.zeros((16,), jnp.float32)
    for k in range(WB // ZR):
        pltpu.sync_copy(xf32.at[0, pl.ds(0, ZR)],
                        accum.at[pl.ds(s * WB + k * ZR, ZR)])

    @pl.when(s == NS - 1)
    def _zero_tail():
        pltpu.sync_copy(xf32.at[0, pl.ds(0, WBT)],
                        accum.at[pl.ds(NS * WB, WBT)])

    plsc.subcore_barrier()

    pltpu.sync_copy(idx_hbm.at[s], idx_v)

    pltpu.async_copy(x_hbm.at[c, pl.ds(s * EPH, GSH)], xint.at[0], gsem)

    def group(i, carry):
        par = lax.rem(i, 2)
        nxt = lax.rem(i + 1, 2)

        @pl.when(i + 1 < G)
        def _prefetch():
            pltpu.async_copy(x_hbm.at[c, pl.ds(s * EPH + (i + 1) * GSH, GSH)],
                             xint.at[nxt], gsem)

        pltpu.make_async_copy(x_hbm.at[c, pl.ds(s * EPH, GSH)],
                              xint.at[par], gsem).wait()

        @pl.when(i >= 1)
        def _drain_prev_scatter():
            pltpu.make_async_copy(xf32.at[par],
                                  accum.at[pl.ds(0, GS)], ssem).wait()

        for u in range(GSH):
            for k in range(DH // 16):
                v = xint[par, u, pl.ds(k * 16, 16)]
                xf32[par, u, pl.ds(k * 16, 16)] = (
                    plsc.bitcast(v << 16, jnp.float32))
                xf32[par, u + GSH, pl.ds(k * 16, 16)] = (
                    plsc.bitcast(v & himask, jnp.float32))

        pltpu.async_copy(xf32.at[par], accum.at[idx_v.at[i]], ssem, add=True)
        return carry

    lax.fori_loop(0, G, group, 0)
    pltpu.make_async_copy(xf32.at[0], accum.at[pl.ds(0, GS)], ssem).wait()
    plsc.subcore_barrier()

    pltpu.sync_copy(accum.at[pl.ds(s * WB, WB)],
                    out_hbm.at[c, pl.ds(s * WB, WB)])

    @pl.when(s == NS - 1)
    def _write_tail():
        pltpu.sync_copy(accum.at[pl.ds(NS * WB, WBT)],
                        out_hbm.at[c, pl.ds(NS * WB, WBT)])


def _mlp_body(x2_ref, w1_ref, wa0_ref, wb0_ref, wa1_ref, wb1_ref,
              wa2_ref, wb2_ref, out_ref):
    inv_sqrt2 = jnp.float32(0.7071067811865476)
    a = x2_ref[0]
    b = x2_ref[1]
    w1 = w1_ref[...]
    x = jax.nn.silu(
        jnp.dot(a, w1[:DH, :], preferred_element_type=jnp.float32)
        + jnp.dot(b, w1[DH:, :], preferred_element_type=jnp.float32))
    for wa_ref, wb_ref in ((wa0_ref, wb0_ref), (wa1_ref, wb1_ref),
                           (wa2_ref, wb2_ref)):
        y = jax.nn.silu(jnp.dot(x, wa_ref[...],
                                preferred_element_type=jnp.float32))
        y = jax.nn.silu(jnp.dot(y, wb_ref[...],
                                preferred_element_type=jnp.float32))
        x = (x + y) * inv_sqrt2
    out_ref[...] = x


def _paired_edge_order() -> np.ndarray:
    t = np.arange(E // 2)
    e_lo = (t // HB) * EBLK + t % HB
    e_hi = e_lo + HB
    order = np.empty((NS, G, GS), dtype=np.int32)
    order[:, :, :GSH] = e_lo.reshape(NS, G, GSH)
    order[:, :, GSH:] = e_hi.reshape(NS, G, GSH)
    return order.reshape(-1)


def kernel(h, m, rbf, id_j, W_rbf, W_dense1,
           W_res0a, W_res0b, W_res1a, W_res1b, W_res2a, W_res2b, scale):
    w_rbf_s = W_rbf * scale

    xsplit = pl.pallas_call(
        _edge_fma_body,
        grid=(E // EBLK,),
        in_specs=[
            pl.BlockSpec((EBLK, D), lambda i: (i, 0)),
            pl.BlockSpec((EBLK, R), lambda i: (i, 0)),
            pl.BlockSpec((R, D), lambda i: (0, 0)),
        ],
        out_specs=pl.BlockSpec((NC, HB, DH), lambda i: (0, i, 0)),
        out_shape=jax.ShapeDtypeStruct((NC, E // 2, DH), jnp.int32),
    )(m, rbf, w_rbf_s)

    idx3 = (id_j.astype(jnp.int32)[jnp.asarray(_paired_edge_order())]
            .reshape(NS, G, GS))
    seg = pl.kernel(
        _seg_sum_body,
        out_type=jax.ShapeDtypeStruct((NC, N, DH), jnp.float32),
        mesh=plsc.VectorSubcoreMesh(core_axis_name="c", subcore_axis_name="s"),
        compiler_params=pltpu.CompilerParams(needs_layout_passes=False),
        scratch_types=[
            pltpu.VMEM((G, GS), jnp.int32),
            pltpu.VMEM((2, GSH, DH), jnp.int32),
            pltpu.VMEM((2, GS, DH), jnp.float32),
            pltpu.SemaphoreType.DMA,
            pltpu.SemaphoreType.DMA,
            pltpu.VMEM_SHARED((N, DH), jnp.float32),
        ],
    )
    x2 = seg(xsplit, idx3)

    wspec = pl.BlockSpec((D, D), lambda i: (0, 0))
    out = pl.pallas_call(
        _mlp_body,
        grid=(N // NBLK,),
        in_specs=[
            pl.BlockSpec((NC, NBLK, DH), lambda i: (0, i, 0)),
            wspec, wspec, wspec, wspec, wspec, wspec, wspec,
        ],
        out_specs=pl.BlockSpec((NBLK, D), lambda i: (i, 0)),
        out_shape=jax.ShapeDtypeStruct((N, D), jnp.float32),
    )(x2, W_dense1, W_res0a, W_res0b, W_res1a, W_res1b, W_res2a, W_res2b)
    return out


# speedup vs baseline: 1.2273x; 1.2273x over previous
import functools

import jax
import jax.numpy as jnp
from jax import lax
from jax.experimental import pallas as pl
from jax.experimental.pallas import tpu as pltpu
from jax.experimental.pallas import tpu_sc as plsc

E = 160000
NCHUNK = 2
EC = E // NCHUNK
N = 10000
D = 256
DH = 128
R = 16

NC = 2
NS = 16
EP = EC // NS
GS = 40
G = EP // GS
WB = 624
WBT = N - NS * WB
ZR = 48

EBLK = 8000
NBLK = 1000


def _edge_fma_body(m_ref, rbf_ref, w_ref, out_ref):
    mlp = jnp.dot(rbf_ref[...], w_ref[...], preferred_element_type=jnp.float32)
    y = m_ref[...] * mlp
    out_ref[0] = y[:, :DH]
    out_ref[1] = y[:, DH:]


def _seg_sum_core(x_hbm, idx_hbm, out_hbm, idx_v, xbuf, gsem, ssem, accum):
    c = lax.axis_index("c")
    s = lax.axis_index("s")

    pltpu.sync_copy(idx_hbm.at[s], idx_v)

    gd = [None] * G
    sd = [None] * G
    gd[0] = pltpu.async_copy(x_hbm.at[c, pl.ds(s * EP, GS)], xbuf.at[0], gsem)
    for i in range(G):
        if i + 1 < G:
            if i >= 1:
                sd[i - 1].wait()
            gd[i + 1] = pltpu.async_copy(
                x_hbm.at[c, pl.ds(s * EP + (i + 1) * GS, GS)],
                xbuf.at[(i + 1) % 2], gsem)
        gd[i].wait()
        sd[i] = pltpu.async_copy(xbuf.at[i % 2],
                                 accum.at[idx_v.at[i]], ssem, add=True)
    sd[G - 2].wait()
    sd[G - 1].wait()
    plsc.subcore_barrier()

    pltpu.sync_copy(accum.at[pl.ds(s * WB, WB)],
                    out_hbm.at[c, pl.ds(s * WB, WB)])

    @pl.when(s == NS - 1)
    def _write_tail():
        pltpu.sync_copy(accum.at[pl.ds(NS * WB, WBT)],
                        out_hbm.at[c, pl.ds(NS * WB, WBT)])


def _seg_sum_first_body(x_hbm, idx_hbm, out_hbm, idx_v, xbuf, gsem, ssem,
                        accum):
    s = lax.axis_index("s")

    for i in range(ZR):
        for k in range(DH // 16):
            xbuf[0, i, pl.ds(k * 16, 16)] = jnp.zeros((16,), jnp.float32)
    for k in range(WB // ZR):
        pltpu.sync_copy(xbuf.at[0, pl.ds(0, ZR)],
                        accum.at[pl.ds(s * WB + k * ZR, ZR)])

    @pl.when(s == NS - 1)
    def _zero_tail():
        pltpu.sync_copy(xbuf.at[0, pl.ds(0, WBT)],
                        accum.at[pl.ds(NS * WB, WBT)])

    plsc.subcore_barrier()
    _seg_sum_core(x_hbm, idx_hbm, out_hbm, idx_v, xbuf, gsem, ssem, accum)


def _seg_sum_next_body(x_hbm, idx_hbm, prev_hbm, out_hbm, idx_v, xbuf, gsem,
                       ssem, accum):
    c = lax.axis_index("c")
    s = lax.axis_index("s")

    pltpu.sync_copy(prev_hbm.at[c, pl.ds(s * WB, WB)],
                    accum.at[pl.ds(s * WB, WB)])

    @pl.when(s == NS - 1)
    def _seed_tail():
        pltpu.sync_copy(prev_hbm.at[c, pl.ds(NS * WB, WBT)],
                        accum.at[pl.ds(NS * WB, WBT)])

    plsc.subcore_barrier()
    _seg_sum_core(x_hbm, idx_hbm, out_hbm, idx_v, xbuf, gsem, ssem, accum)


def _mlp_body(x2_ref, w1_ref, wa0_ref, wb0_ref, wa1_ref, wb1_ref,
              wa2_ref, wb2_ref, out_ref):
    inv_sqrt2 = jnp.float32(0.7071067811865476)
    a = x2_ref[0]
    b = x2_ref[1]
    w1 = w1_ref[...]
    x = jax.nn.silu(
        jnp.dot(a, w1[:DH, :], preferred_element_type=jnp.float32)
        + jnp.dot(b, w1[DH:, :], preferred_element_type=jnp.float32))
    for wa_ref, wb_ref in ((wa0_ref, wb0_ref), (wa1_ref, wb1_ref),
                           (wa2_ref, wb2_ref)):
        y = jax.nn.silu(jnp.dot(x, wa_ref[...],
                                preferred_element_type=jnp.float32))
        y = jax.nn.silu(jnp.dot(y, wb_ref[...],
                                preferred_element_type=jnp.float32))
        x = (x + y) * inv_sqrt2
    out_ref[...] = x


_SC_SCRATCH = [
    pltpu.VMEM((G, GS), jnp.int32),
    pltpu.VMEM((2, GS, DH), jnp.float32),
    pltpu.SemaphoreType.DMA,
    pltpu.SemaphoreType.DMA,
    pltpu.VMEM_SHARED((N, DH), jnp.float32),
]


def kernel(h, m, rbf, id_j, W_rbf, W_dense1,
           W_res0a, W_res0b, W_res1a, W_res1b, W_res2a, W_res2b, scale):
    w_rbf_s = W_rbf * scale

    def edge_fma(chunk):
        blocks = EC // EBLK
        base = chunk * blocks
        return pl.pallas_call(
            _edge_fma_body,
            grid=(blocks,),
            in_specs=[
                pl.BlockSpec((EBLK, D), lambda i: (base + i, 0)),
                pl.BlockSpec((EBLK, R), lambda i: (base + i, 0)),
                pl.BlockSpec((R, D), lambda i: (0, 0)),
            ],
            out_specs=pl.BlockSpec((NC, EBLK, DH), lambda i: (0, i, 0)),
            out_shape=jax.ShapeDtypeStruct((NC, EC, DH), jnp.float32),
        )(m, rbf, w_rbf_s)

    idx_all = id_j.astype(jnp.int32)
    seg_first = pl.kernel(
        _seg_sum_first_body,
        out_type=jax.ShapeDtypeStruct((NC, N, DH), jnp.float32),
        mesh=plsc.VectorSubcoreMesh(core_axis_name="c", subcore_axis_name="s"),
        scratch_types=_SC_SCRATCH,
    )
    seg_next = pl.kernel(
        _seg_sum_next_body,
        out_type=jax.ShapeDtypeStruct((NC, N, DH), jnp.float32),
        mesh=plsc.VectorSubcoreMesh(core_axis_name="c", subcore_axis_name="s"),
        scratch_types=_SC_SCRATCH,
    )

    x2 = None
    for chunk in range(NCHUNK):
        xc = edge_fma(chunk)
        idxc = idx_all[chunk * EC:(chunk + 1) * EC].reshape(NS, G, GS)
        if x2 is None:
            x2 = seg_first(xc, idxc)
        else:
            x2 = seg_next(xc, idxc, x2)

    wspec = pl.BlockSpec((D, D), lambda i: (0, 0))
    out = pl.pallas_call(
        _mlp_body,
        grid=(N // NBLK,),
        in_specs=[
            pl.BlockSpec((NC, NBLK, DH), lambda i: (0, i, 0)),
            wspec, wspec, wspec, wspec, wspec, wspec, wspec,
        ],
        out_specs=pl.BlockSpec((NBLK, D), lambda i: (i, 0)),
        out_shape=jax.ShapeDtypeStruct((N, D), jnp.float32),
    )(x2, W_dense1, W_res0a, W_res0b, W_res1a, W_res1b, W_res2a, W_res2b)
    return out


# speedup vs baseline: 1.4211x; 1.1579x over previous
import functools

import jax
import jax.numpy as jnp
from jax import lax
from jax.experimental import pallas as pl
from jax.experimental.pallas import tpu as pltpu
from jax.experimental.pallas import tpu_sc as plsc

E = 160000
N = 10000
D = 256
DH = 128
R = 16

NC = 2
NS = 16
EP = E // NS
GS = 80
G = EP // GS
WB = 624
WBT = N - NS * WB
ZR = 48

EBLK = 8000
NBLK = 1000


def _edge_fma_body(m_ref, rbf_ref, w_ref, out_ref):
    mlp = jnp.dot(rbf_ref[...], w_ref[...], preferred_element_type=jnp.float32)
    y = m_ref[...] * mlp
    out_ref[0] = y[:, :DH]
    out_ref[1] = y[:, DH:]


def _seg_sum_body(x_hbm, idx_hbm, out_hbm, idx_v, xbuf, gsem, ssem, accum):
    c = lax.axis_index("c")
    s = lax.axis_index("s")

    for i in range(ZR):
        for k in range(DH // 16):
            xbuf[0, i, pl.ds(k * 16, 16)] = jnp.zeros((16,), jnp.float32)
    for k in range(WB // ZR):
        pltpu.sync_copy(xbuf.at[0, pl.ds(0, ZR)],
                        accum.at[pl.ds(s * WB + k * ZR, ZR)])

    @pl.when(s == NS - 1)
    def _zero_tail():
        pltpu.sync_copy(xbuf.at[0, pl.ds(0, WBT)],
                        accum.at[pl.ds(NS * WB, WBT)])

    plsc.subcore_barrier()

    pltpu.sync_copy(idx_hbm.at[s], idx_v)

    def gather(i, b):
        return pltpu.async_copy(
            x_hbm.at[c, pl.ds(s * EP + i * GS, GS)], xbuf.at[b], gsem)

    gd = [None] * G
    sd = [None] * G
    gd[0] = gather(0, 0)
    gd[1] = gather(1, 1)
    for i in range(G):
        gd[i].wait()
        sd[i] = pltpu.async_copy(xbuf.at[i % 3],
                                 accum.at[idx_v.at[i]], ssem, add=True)
        if i + 2 < G:
            if i >= 1:
                sd[i - 1].wait()
            gd[i + 2] = gather(i + 2, (i + 2) % 3)
    sd[G - 2].wait()
    sd[G - 1].wait()
    plsc.subcore_barrier()

    pltpu.sync_copy(accum.at[pl.ds(s * WB, WB)],
                    out_hbm.at[c, pl.ds(s * WB, WB)])

    @pl.when(s == NS - 1)
    def _write_tail():
        pltpu.sync_copy(accum.at[pl.ds(NS * WB, WBT)],
                        out_hbm.at[c, pl.ds(NS * WB, WBT)])


def _mlp_body(x2_ref, w1_ref, wa0_ref, wb0_ref, wa1_ref, wb1_ref,
              wa2_ref, wb2_ref, out_ref):
    inv_sqrt2 = jnp.float32(0.7071067811865476)
    a = x2_ref[0]
    b = x2_ref[1]
    w1 = w1_ref[...]
    x = jax.nn.silu(
        jnp.dot(a, w1[:DH, :], preferred_element_type=jnp.float32)
        + jnp.dot(b, w1[DH:, :], preferred_element_type=jnp.float32))
    for wa_ref, wb_ref in ((wa0_ref, wb0_ref), (wa1_ref, wb1_ref),
                           (wa2_ref, wb2_ref)):
        y = jax.nn.silu(jnp.dot(x, wa_ref[...],
                                preferred_element_type=jnp.float32))
        y = jax.nn.silu(jnp.dot(y, wb_ref[...],
                                preferred_element_type=jnp.float32))
        x = (x + y) * inv_sqrt2
    out_ref[...] = x


def kernel(h, m, rbf, id_j, W_rbf, W_dense1,
           W_res0a, W_res0b, W_res1a, W_res1b, W_res2a, W_res2b, scale):
    w_rbf_s = W_rbf * scale

    xsplit = pl.pallas_call(
        _edge_fma_body,
        grid=(E // EBLK,),
        in_specs=[
            pl.BlockSpec((EBLK, D), lambda i: (i, 0)),
            pl.BlockSpec((EBLK, R), lambda i: (i, 0)),
            pl.BlockSpec((R, D), lambda i: (0, 0)),
        ],
        out_specs=pl.BlockSpec((NC, EBLK, DH), lambda i: (0, i, 0)),
        out_shape=jax.ShapeDtypeStruct((NC, E, DH), jnp.float32),
    )(m, rbf, w_rbf_s)

    idx3 = id_j.astype(jnp.int32).reshape(NS, G, GS)
    seg = pl.kernel(
        _seg_sum_body,
        out_type=jax.ShapeDtypeStruct((NC, N, DH), jnp.float32),
        mesh=plsc.VectorSubcoreMesh(core_axis_name="c", subcore_axis_name="s"),
        scratch_types=[
            pltpu.VMEM((G, GS), jnp.int32),
            pltpu.VMEM((3, GS, DH), jnp.float32),
            pltpu.SemaphoreType.DMA,
            pltpu.SemaphoreType.DMA,
            pltpu.VMEM_SHARED((N, DH), jnp.float32),
        ],
    )
    x2 = seg(xsplit, idx3)

    wspec = pl.BlockSpec((D, D), lambda i: (0, 0))
    out = pl.pallas_call(
        _mlp_body,
        grid=(N // NBLK,),
        in_specs=[
            pl.BlockSpec((NC, NBLK, DH), lambda i: (0, i, 0)),
            wspec, wspec, wspec, wspec, wspec, wspec, wspec,
        ],
        out_specs=pl.BlockSpec((NBLK, D), lambda i: (i, 0)),
        out_shape=jax.ShapeDtypeStruct((N, D), jnp.float32),
    )(x2, W_dense1, W_res0a, W_res0b, W_res1a, W_res1b, W_res2a, W_res2b)
    return out


# speedup vs baseline: 1.4675x; 1.0326x over previous
import functools

import jax
import jax.numpy as jnp
from jax import lax
from jax.experimental import pallas as pl
from jax.experimental.pallas import tpu as pltpu
from jax.experimental.pallas import tpu_sc as plsc

E = 160000
N = 10000
D = 256
DH = 128
R = 16

NC = 2
NS = 16
EP = E // NS
GS = 80
G = EP // GS
WB = 624
WBT = N - NS * WB
ZR = 48

EBLK = 8000
NBLK = 1000


def _edge_fma_body(m_ref, rbf_ref, w_ref, out_ref):
    mlp = jnp.dot(rbf_ref[...], w_ref[...], preferred_element_type=jnp.float32)
    y = m_ref[...] * mlp
    out_ref[0] = y[:, :DH]
    out_ref[1] = y[:, DH:]


def _seg_sum_body(x_hbm, idx_hbm, out_hbm, idx_v, xbuf, gsem, ssem0, ssem1,
                  ssem2, accum):
    c = lax.axis_index("c")
    s = lax.axis_index("s")
    ssems = (ssem0, ssem1, ssem2)

    for i in range(ZR):
        for k in range(DH // 16):
            xbuf[0, i, pl.ds(k * 16, 16)] = jnp.zeros((16,), jnp.float32)
    for k in range(WB // ZR):
        pltpu.sync_copy(xbuf.at[0, pl.ds(0, ZR)],
                        accum.at[pl.ds(s * WB + k * ZR, ZR)])

    @pl.when(s == NS - 1)
    def _zero_tail():
        pltpu.sync_copy(xbuf.at[0, pl.ds(0, WBT)],
                        accum.at[pl.ds(NS * WB, WBT)])

    plsc.subcore_barrier()

    pltpu.sync_copy(idx_hbm.at[s], idx_v)

    def gather(i, b):
        return pltpu.async_copy(
            x_hbm.at[c, pl.ds(s * EP + i * GS, GS)], xbuf.at[b], gsem)

    gd = [None] * G
    sd = [None] * G
    gd[0] = gather(0, 0)
    gd[1] = gather(1, 1)
    for i in range(G):
        gd[i].wait()
        sd[i] = pltpu.async_copy(xbuf.at[i % 3],
                                 accum.at[idx_v.at[i]], ssems[i % 3],
                                 add=True)
        if i + 2 < G:
            if i >= 1:
                sd[i - 1].wait()
            gd[i + 2] = gather(i + 2, (i + 2) % 3)
    sd[G - 2].wait()
    sd[G - 1].wait()
    plsc.subcore_barrier()

    pltpu.sync_copy(accum.at[pl.ds(s * WB, WB)],
                    out_hbm.at[c, pl.ds(s * WB, WB)])

    @pl.when(s == NS - 1)
    def _write_tail():
        pltpu.sync_copy(accum.at[pl.ds(NS * WB, WBT)],
                        out_hbm.at[c, pl.ds(NS * WB, WBT)])


def _mlp_body(x2_ref, w1_ref, wa0_ref, wb0_ref, wa1_ref, wb1_ref,
              wa2_ref, wb2_ref, out_ref):
    inv_sqrt2 = jnp.float32(0.7071067811865476)
    a = x2_ref[0]
    b = x2_ref[1]
    w1 = w1_ref[...]
    x = jax.nn.silu(
        jnp.dot(a, w1[:DH, :], preferred_element_type=jnp.float32)
        + jnp.dot(b, w1[DH:, :], preferred_element_type=jnp.float32))
    for wa_ref, wb_ref in ((wa0_ref, wb0_ref), (wa1_ref, wb1_ref),
                           (wa2_ref, wb2_ref)):
        y = jax.nn.silu(jnp.dot(x, wa_ref[...],
                                preferred_element_type=jnp.float32))
        y = jax.nn.silu(jnp.dot(y, wb_ref[...],
                                preferred_element_type=jnp.float32))
        x = (x + y) * inv_sqrt2
    out_ref[...] = x


def kernel(h, m, rbf, id_j, W_rbf, W_dense1,
           W_res0a, W_res0b, W_res1a, W_res1b, W_res2a, W_res2b, scale):
    w_rbf_s = W_rbf * scale

    xsplit = pl.pallas_call(
        _edge_fma_body,
        grid=(E // EBLK,),
        in_specs=[
            pl.BlockSpec((EBLK, D), lambda i: (i, 0)),
            pl.BlockSpec((EBLK, R), lambda i: (i, 0)),
            pl.BlockSpec((R, D), lambda i: (0, 0)),
        ],
        out_specs=pl.BlockSpec((NC, EBLK, DH), lambda i: (0, i, 0)),
        out_shape=jax.ShapeDtypeStruct((NC, E, DH), jnp.float32),
    )(m, rbf, w_rbf_s)

    idx3 = id_j.astype(jnp.int32).reshape(NS, G, GS)
    seg = pl.kernel(
        _seg_sum_body,
        out_type=jax.ShapeDtypeStruct((NC, N, DH), jnp.float32),
        mesh=plsc.VectorSubcoreMesh(core_axis_name="c", subcore_axis_name="s"),
        scratch_types=[
            pltpu.VMEM((G, GS), jnp.int32),
            pltpu.VMEM((3, GS, DH), jnp.float32),
            pltpu.SemaphoreType.DMA,
            pltpu.SemaphoreType.DMA,
            pltpu.SemaphoreType.DMA,
            pltpu.SemaphoreType.DMA,
            pltpu.VMEM_SHARED((N, DH), jnp.float32),
        ],
    )
    x2 = seg(xsplit, idx3)

    wspec = pl.BlockSpec((D, D), lambda i: (0, 0))
    out = pl.pallas_call(
        _mlp_body,
        grid=(N // NBLK,),
        in_specs=[
            pl.BlockSpec((NC, NBLK, DH), lambda i: (0, i, 0)),
            wspec, wspec, wspec, wspec, wspec, wspec, wspec,
        ],
        out_specs=pl.BlockSpec((NBLK, D), lambda i: (i, 0)),
        out_shape=jax.ShapeDtypeStruct((N, D), jnp.float32),
    )(x2, W_dense1, W_res0a, W_res0b, W_res1a, W_res1b, W_res2a, W_res2b)
    return out
